# baseline (speedup 1.0000x reference)
import functools

import jax
import jax.numpy as jnp
from jax.experimental import pallas as pl
from jax.experimental.pallas import tpu as pltpu

TH, TW = 15, 15
PAD_HT = (TH - 1) // 2
PAD_WL = (TW - 1) // 2
MARGIN = 0.25
GAMMA = 256.0
NEG = -1e30


def _loss_body(x_ref, m_ref, o_ref, mp_a, sp_a, mn_a, sn_a, *,
               nrows, W, r0, r1, c0, c1):
    i = pl.program_id(0)

    @pl.when(i == 0)
    def _init():
        mp_a[...] = jnp.full((1, W), NEG, jnp.float32)
        sp_a[...] = jnp.zeros((1, W), jnp.float32)
        mn_a[...] = jnp.full((1, W), NEG, jnp.float32)
        sn_a[...] = jnp.zeros((1, W), jnp.float32)

    x = x_ref[...]
    tm = m_ref[...]

    rid = jax.lax.broadcasted_iota(jnp.int32, (nrows, 1), 0) + i * nrows
    row_ok = (rid >= r0) & (rid < r1)
    cid = jax.lax.broadcasted_iota(jnp.int32, (1, W), 1)
    col_ok = (cid >= c0) & (cid < c1)

    finite = jnp.isfinite(x)
    okp = tm & finite
    ok = finite & (tm | (row_ok & col_ok))
    okn = ok & (~okp)

    u = jnp.where(tm, -x, x)
    a = jnp.where(tm, 1.0 + MARGIN, MARGIN)
    b = jnp.where(tm, 1.0 - MARGIN, -MARGIN)
    l = (jnp.maximum(u + a, 0.0) * (u + b)) * GAMMA
    l = jnp.where(ok, l, NEG)

    bmp = jnp.max(jnp.where(okp, l, NEG), axis=0, keepdims=True)
    bmn = jnp.max(jnp.where(okp, NEG, l), axis=0, keepdims=True)
    mp_old = mp_a[...]
    mn_old = mn_a[...]
    mp = jnp.maximum(mp_old, bmp)
    mn = jnp.maximum(mn_old, bmn)

    e = jnp.exp(l - jnp.where(tm, mp, mn))
    sp_add = jnp.sum(jnp.where(okp, e, 0.0), axis=0, keepdims=True)
    sn_add = jnp.sum(jnp.where(okn, e, 0.0), axis=0, keepdims=True)

    mp_a[...] = mp
    sp_a[...] = sp_a[...] * jnp.exp(mp_old - mp) + sp_add
    mn_a[...] = mn
    sn_a[...] = sn_a[...] * jnp.exp(mn_old - mn) + sn_add

    @pl.when(i == pl.num_programs(0) - 1)
    def _fin():
        mp_c = mp_a[...]
        mn_c = mn_a[...]
        Mp = jnp.max(mp_c)
        Mn = jnp.max(mn_c)
        Sp = jnp.sum(sp_a[...] * jnp.exp(mp_c - Mp))
        Sn = jnp.sum(sn_a[...] * jnp.exp(mn_c - Mn))
        z = Mp + jnp.log(Sp) + Mn + jnp.log(Sn)
        o_ref[0, 0] = jnp.maximum(z, 0.0) + jnp.log1p(jnp.exp(-jnp.abs(z)))


def kernel(ypred, truthMask):
    B, H, W = ypred.shape
    mh, mw = truthMask.shape[-2], truthMask.shape[-1]
    r0 = PAD_HT - 1
    r1 = min(PAD_HT - TH + mh + 2, H)
    c0 = PAD_WL - 1
    c1 = min(PAD_WL - TW + mw + 2, W)

    x = ypred.reshape(H, W)
    tm = truthMask.reshape(H, W)

    nrows = 128 if H % 128 == 0 else H
    grid = H // nrows

    out = pl.pallas_call(
        functools.partial(
            _loss_body, nrows=nrows, W=W, r0=r0, r1=r1, c0=c0, c1=c1
        ),
        grid=(grid,),
        in_specs=[
            pl.BlockSpec((nrows, W), lambda i: (i, 0)),
            pl.BlockSpec((nrows, W), lambda i: (i, 0)),
        ],
        out_specs=pl.BlockSpec(
            (1, 1), lambda i: (0, 0), memory_space=pltpu.SMEM
        ),
        out_shape=jax.ShapeDtypeStruct((1, 1), jnp.float32),
        scratch_shapes=[
            pltpu.VMEM((1, W), jnp.float32),
            pltpu.VMEM((1, W), jnp.float32),
            pltpu.VMEM((1, W), jnp.float32),
            pltpu.VMEM((1, W), jnp.float32),
        ],
        compiler_params=pltpu.CompilerParams(
            dimension_semantics=("arbitrary",),
        ),
    )(x, tm)
    return out.reshape(B)


# speedup vs baseline: 1.7548x; 1.7548x over previous
import functools

import jax
import jax.numpy as jnp
from jax.experimental import pallas as pl
from jax.experimental.pallas import tpu as pltpu

TH, TW = 15, 15
PAD_HT = (TH - 1) // 2
PAD_WL = (TW - 1) // 2
MARGIN = 0.25
GAMMA = 256.0
NEG = -1e30
PEN = -1e35


def _loss_body(x_ref, m_ref, o_ref, acc, *, nrows, W, r0, r1, c0, c1):
    i = pl.program_id(0)

    @pl.when(i == 0)
    def _init():
        acc[0] = NEG
        acc[1] = 0.0
        acc[2] = NEG
        acc[3] = 0.0

    x = x_ref[...]
    tm = m_ref[...]

    rid = jax.lax.broadcasted_iota(jnp.int32, (nrows, 1), 0) + i * nrows
    rowpen = jnp.where((rid >= r0) & (rid < r1), 0.0, PEN)
    cid = jax.lax.broadcasted_iota(jnp.int32, (1, W), 1)
    colpen = jnp.where((cid >= c0) & (cid < c1), 0.0, PEN)

    y = x * GAMMA
    lp = jnp.maximum(GAMMA * (1.0 + MARGIN) - y, 0.0) * ((1.0 - MARGIN) - x)
    ln = jnp.maximum(y + GAMMA * MARGIN, 0.0) * (x - MARGIN) + (rowpen + colpen)
    l = jnp.where(tm, lp, ln)

    mp_old = acc[0]
    mn_old = acc[2]
    mp = jnp.maximum(mp_old, jnp.max(jnp.where(tm, l, NEG)))
    mn = jnp.maximum(mn_old, jnp.max(jnp.where(tm, NEG, l)))

    e = jnp.exp(l - jnp.where(tm, mp, mn))
    sp_add = jnp.sum(jnp.where(tm, e, 0.0))
    sn_add = jnp.sum(e) - sp_add

    acc[0] = mp
    acc[1] = acc[1] * jnp.exp(mp_old - mp) + sp_add
    acc[2] = mn
    acc[3] = acc[3] * jnp.exp(mn_old - mn) + sn_add

    @pl.when(i == pl.num_programs(0) - 1)
    def _fin():
        z = acc[0] + jnp.log(acc[1]) + acc[2] + jnp.log(acc[3])
        o_ref[0, 0] = jnp.maximum(z, 0.0) + jnp.log1p(jnp.exp(-jnp.abs(z)))


def kernel(ypred, truthMask):
    B, H, W = ypred.shape
    mh, mw = truthMask.shape[-2], truthMask.shape[-1]
    r0 = PAD_HT - 1
    r1 = min(PAD_HT - TH + mh + 2, H)
    c0 = PAD_WL - 1
    c1 = min(PAD_WL - TW + mw + 2, W)

    x = ypred.reshape(H, W)
    tm = truthMask.reshape(H, W)

    nrows = 128 if H % 128 == 0 else H
    grid = H // nrows

    out = pl.pallas_call(
        functools.partial(
            _loss_body, nrows=nrows, W=W, r0=r0, r1=r1, c0=c0, c1=c1
        ),
        grid=(grid,),
        in_specs=[
            pl.BlockSpec((nrows, W), lambda i: (i, 0)),
            pl.BlockSpec((nrows, W), lambda i: (i, 0)),
        ],
        out_specs=pl.BlockSpec(
            (1, 1), lambda i: (0, 0), memory_space=pltpu.SMEM
        ),
        out_shape=jax.ShapeDtypeStruct((1, 1), jnp.float32),
        scratch_shapes=[pltpu.SMEM((4,), jnp.float32)],
        compiler_params=pltpu.CompilerParams(
            dimension_semantics=("arbitrary",),
        ),
    )(x, tm)
    return out.reshape(B)


# speedup vs baseline: 1.8744x; 1.0681x over previous
import functools

import jax
import jax.numpy as jnp
from jax.experimental import pallas as pl
from jax.experimental.pallas import tpu as pltpu

TH, TW = 15, 15
PAD_HT = (TH - 1) // 2
PAD_WL = (TW - 1) // 2
MARGIN = 0.25
GAMMA = 256.0
NEG = -1e30
PEN = -1e35


def _loss_body(x_ref, m_ref, o_ref, mp_a, sp_a, mn_a, sn_a, *,
               nrows, W, r0, r1, c0, c1):
    i = pl.program_id(0)
    nsub = nrows // 8

    @pl.when(i == 0)
    def _init():
        mp_a[...] = jnp.full((8, W), NEG, jnp.float32)
        sp_a[...] = jnp.zeros((8, W), jnp.float32)
        mn_a[...] = jnp.full((8, W), NEG, jnp.float32)
        sn_a[...] = jnp.zeros((8, W), jnp.float32)

    x = x_ref[...]
    tm = m_ref[...]

    rid = jax.lax.broadcasted_iota(jnp.int32, (nrows, 1), 0) + i * nrows
    rowpen = jnp.where((rid >= r0) & (rid < r1), 0.0, PEN)
    cid = jax.lax.broadcasted_iota(jnp.int32, (1, W), 1)
    colpen = jnp.where((cid >= c0) & (cid < c1), 0.0, PEN)

    y = x * GAMMA
    lp = jnp.maximum(GAMMA * (1.0 + MARGIN) - y, 0.0) * ((1.0 - MARGIN) - x)
    ln = (jnp.maximum(y + GAMMA * MARGIN, 0.0) * (x - MARGIN)
          + (rowpen + colpen))
    lP = jnp.where(tm, lp, PEN).reshape(nsub, 8, W)
    lN = jnp.where(tm, PEN, ln).reshape(nsub, 8, W)

    mp_old = mp_a[...]
    mn_old = mn_a[...]
    mp = jnp.maximum(mp_old, jnp.max(lP, axis=0))
    mn = jnp.maximum(mn_old, jnp.max(lN, axis=0))

    ep = jnp.exp(lP - mp[None, :, :])
    en = jnp.exp(lN - mn[None, :, :])

    mp_a[...] = mp
    sp_a[...] = sp_a[...] * jnp.exp(mp_old - mp) + jnp.sum(ep, axis=0)
    mn_a[...] = mn
    sn_a[...] = sn_a[...] * jnp.exp(mn_old - mn) + jnp.sum(en, axis=0)

    @pl.when(i == pl.num_programs(0) - 1)
    def _fin():
        mp_c = mp_a[...]
        mn_c = mn_a[...]
        Mp = jnp.max(mp_c)
        Mn = jnp.max(mn_c)
        Sp = jnp.sum(sp_a[...] * jnp.exp(mp_c - Mp))
        Sn = jnp.sum(sn_a[...] * jnp.exp(mn_c - Mn))
        z = Mp + jnp.log(Sp) + Mn + jnp.log(Sn)
        o_ref[0, 0] = jnp.maximum(z, 0.0) + jnp.log1p(jnp.exp(-jnp.abs(z)))


def kernel(ypred, truthMask):
    B, H, W = ypred.shape
    mh, mw = truthMask.shape[-2], truthMask.shape[-1]
    r0 = PAD_HT - 1
    r1 = min(PAD_HT - TH + mh + 2, H)
    c0 = PAD_WL - 1
    c1 = min(PAD_WL - TW + mw + 2, W)

    x = ypred.reshape(H, W)
    tm = truthMask.reshape(H, W)

    nrows = 128 if H % 128 == 0 else H
    grid = H // nrows

    out = pl.pallas_call(
        functools.partial(
            _loss_body, nrows=nrows, W=W, r0=r0, r1=r1, c0=c0, c1=c1
        ),
        grid=(grid,),
        in_specs=[
            pl.BlockSpec((nrows, W), lambda i: (i, 0)),
            pl.BlockSpec((nrows, W), lambda i: (i, 0)),
        ],
        out_specs=pl.BlockSpec(
            (1, 1), lambda i: (0, 0), memory_space=pltpu.SMEM
        ),
        out_shape=jax.ShapeDtypeStruct((1, 1), jnp.float32),
        scratch_shapes=[
            pltpu.VMEM((8, W), jnp.float32),
            pltpu.VMEM((8, W), jnp.float32),
            pltpu.VMEM((8, W), jnp.float32),
            pltpu.VMEM((8, W), jnp.float32),
        ],
        compiler_params=pltpu.CompilerParams(
            dimension_semantics=("arbitrary",),
        ),
    )(x, tm)
    return out.reshape(B)


# speedup vs baseline: 1.8848x; 1.0055x over previous
import functools

import jax
import jax.numpy as jnp
from jax.experimental import pallas as pl
from jax.experimental.pallas import tpu as pltpu

TH, TW = 15, 15
PAD_HT = (TH - 1) // 2
PAD_WL = (TW - 1) // 2
MARGIN = 0.25
GAMMA = 256.0
NEG = -1e30
PEN = -1e35


def _loss_body(x_ref, m_ref, o_ref, mp_a, sp_a, mn_a, sn_a, *,
               nrows, W, r0, r1, c0, c1):
    i = pl.program_id(0)
    nsub = nrows // 8

    @pl.when(i == 0)
    def _init():
        mp_a[...] = jnp.full((8, W), NEG, jnp.float32)
        sp_a[...] = jnp.zeros((8, W), jnp.float32)
        mn_a[...] = jnp.full((8, W), NEG, jnp.float32)
        sn_a[...] = jnp.zeros((8, W), jnp.float32)

    x = x_ref[...]
    tm = m_ref[...]

    rid = jax.lax.broadcasted_iota(jnp.int32, (nrows, 1), 0) + i * nrows
    rowpen = jnp.where((rid >= r0) & (rid < r1), 0.0, PEN)
    cid = jax.lax.broadcasted_iota(jnp.int32, (1, W), 1)
    colpen = jnp.where((cid >= c0) & (cid < c1), 0.0, PEN)

    y = x * GAMMA
    lp = jnp.maximum(GAMMA * (1.0 + MARGIN) - y, 0.0) * ((1.0 - MARGIN) - x)
    ln = (jnp.maximum(y + GAMMA * MARGIN, 0.0) * (x - MARGIN)
          + (rowpen + colpen))
    lP = jnp.where(tm, lp, PEN).reshape(nsub, 8, W)
    lN = jnp.where(tm, PEN, ln).reshape(nsub, 8, W)

    mp_old = mp_a[...]
    mn_old = mn_a[...]
    mp = jnp.maximum(mp_old, jnp.max(lP, axis=0))
    mn = jnp.maximum(mn_old, jnp.max(lN, axis=0))

    ep = jnp.exp(lP - mp[None, :, :])
    en = jnp.exp(lN - mn[None, :, :])

    mp_a[...] = mp
    sp_a[...] = sp_a[...] * jnp.exp(mp_old - mp) + jnp.sum(ep, axis=0)
    mn_a[...] = mn
    sn_a[...] = sn_a[...] * jnp.exp(mn_old - mn) + jnp.sum(en, axis=0)

    @pl.when(i == pl.num_programs(0) - 1)
    def _fin():
        mp_c = mp_a[...]
        mn_c = mn_a[...]
        Mp = jnp.max(mp_c)
        Mn = jnp.max(mn_c)
        Sp = jnp.sum(sp_a[...] * jnp.exp(mp_c - Mp))
        Sn = jnp.sum(sn_a[...] * jnp.exp(mn_c - Mn))
        z = Mp + jnp.log(Sp) + Mn + jnp.log(Sn)
        o_ref[0, 0] = jnp.maximum(z, 0.0) + jnp.log1p(jnp.exp(-jnp.abs(z)))


def kernel(ypred, truthMask):
    B, H, W = ypred.shape
    mh, mw = truthMask.shape[-2], truthMask.shape[-1]
    r0 = PAD_HT - 1
    r1 = min(PAD_HT - TH + mh + 2, H)
    c0 = PAD_WL - 1
    c1 = min(PAD_WL - TW + mw + 2, W)

    x = ypred.reshape(H, W)
    tm = truthMask.reshape(H, W)

    nrows = 256 if H % 256 == 0 else H
    grid = H // nrows

    out = pl.pallas_call(
        functools.partial(
            _loss_body, nrows=nrows, W=W, r0=r0, r1=r1, c0=c0, c1=c1
        ),
        grid=(grid,),
        in_specs=[
            pl.BlockSpec((nrows, W), lambda i: (i, 0)),
            pl.BlockSpec((nrows, W), lambda i: (i, 0)),
        ],
        out_specs=pl.BlockSpec(
            (1, 1), lambda i: (0, 0), memory_space=pltpu.SMEM
        ),
        out_shape=jax.ShapeDtypeStruct((1, 1), jnp.float32),
        scratch_shapes=[
            pltpu.VMEM((8, W), jnp.float32),
            pltpu.VMEM((8, W), jnp.float32),
            pltpu.VMEM((8, W), jnp.float32),
            pltpu.VMEM((8, W), jnp.float32),
        ],
        compiler_params=pltpu.CompilerParams(
            dimension_semantics=("arbitrary",),
        ),
    )(x, tm)
    return out.reshape(B)
